# baseline (speedup 1.0000x reference)
import functools

import jax
import jax.numpy as jnp
from jax import lax
from jax.experimental import pallas as pl
from jax.experimental.pallas import tpu as pltpu
from jax.experimental.pallas import tpu_sc as plsc

_EMBED = 64
_SCALE = 8.0
_ROWS = 16384
_SEQ = 20
_NC, _NS = 2, 16
_NW = _NC * _NS
_RPW = _ROWS // _NW
_HALF = 256
_NBUF = 4
_NTBUF = 2
_NCHUNK = _SEQ * (_RPW // _HALF)

_mesh = plsc.VectorSubcoreMesh(core_axis_name="c", subcore_axis_name="s")


@functools.partial(
    pl.kernel,
    out_type=jax.ShapeDtypeStruct((_SEQ, _EMBED, _ROWS), jnp.float32),
    mesh=_mesh,
    scratch_types=[
        pltpu.VMEM((_SEQ, _RPW), jnp.int32),
        pltpu.VMEM((_NBUF, _HALF, _EMBED), jnp.float32),
        pltpu.VMEM((_NTBUF, _EMBED, _HALF), jnp.float32),
    ] + [pltpu.SemaphoreType.DMA] * (_NBUF + _NTBUF),
    compiler_params=pltpu.CompilerParams(use_tc_tiling_on_sc=False, needs_layout_passes=False),
)
def _embed_lookup(xt_hbm, table_hbm, out_hbm, idx_all, rows, touts, *sems):
    gsems, wsems = sems[:_NBUF], sems[_NBUF:]
    wid = lax.axis_index("s") * _NC + lax.axis_index("c")
    i0 = wid * _RPW

    pltpu.sync_copy(xt_hbm.at[:, pl.ds(i0, _RPW)], idx_all)

    def idx_sl(c):
        j = c // 2
        h = c % 2
        return idx_all.at[j, pl.ds(h * _HALF, _HALF)]

    def start_gather(c, b):
        pltpu.async_copy(table_hbm.at[idx_sl(c)], rows.at[b], gsems[b])

    def wait_gather(b):
        pltpu.make_async_copy(table_hbm.at[idx_sl(0)], rows.at[b], gsems[b]).wait()

    def out_sl(c):
        j = c // 2
        h = c % 2
        return out_hbm.at[j, :, pl.ds(i0 + h * _HALF, _HALF)]

    def start_write(c, t):
        pltpu.async_copy(touts.at[t], out_sl(c), wsems[t])

    def wait_write(t):
        pltpu.make_async_copy(touts.at[t], out_sl(0), wsems[t]).wait()

    lane = lax.iota(jnp.int32, 16)

    start_gather(0, 0)
    start_gather(1, 1)

    @pl.loop(0, _NCHUNK, step=_NBUF)
    def _round(base_c):
        for b in range(_NBUF):
            c = base_c + b
            t = b % _NTBUF
            wait_gather(b)

            @pl.when(c >= _NTBUF)
            def _drain_t():
                wait_write(t)

            @plsc.parallel_loop(0, _EMBED, unroll=2)
            def _tr(e):
                for g in range(_HALF // 16):
                    vals = plsc.load_gather(rows.at[b], [g * 16 + lane,
                                                        jnp.broadcast_to(e, (16,))])
                    touts[t, e, pl.ds(g * 16, 16)] = vals * _SCALE

            start_write(c, t)
            bp = (b + 2) % _NBUF

            @pl.when(c + 2 < _NCHUNK)
            def _prefetch():
                start_gather(c + 2, bp)

    for t in range(_NTBUF):
        wait_write(t)


def kernel(x, table):
    out_t = _embed_lookup(x.T, table)
    return jnp.transpose(out_t, (2, 0, 1))
